# baseline (speedup 1.0000x reference)
import functools

import jax
import jax.numpy as jnp
from jax import lax
from jax.experimental import pallas as pl
from jax.experimental.pallas import tpu as pltpu
from jax.experimental.pallas import tpu_sc as plsc

_NH = 8
_NP = 4
_HD = 32
_NC = 2
_NS = 16
_NW = _NC * _NS
_CH = 512


def _proj_body(x_ref, vpw_ref, vpb_ref, sow_ref, sob_ref, aww_ref, awb_ref,
               tbl_ref, idx_ref, wgt_ref):
    x = x_ref[0]
    nq = x.shape[1]
    f32 = jnp.float32
    vpv = jnp.dot(vpw_ref[...], x, preferred_element_type=f32) + vpb_ref[...]
    offs = jnp.dot(sow_ref[...], x, preferred_element_type=f32) + sob_ref[...]
    awr = jnp.dot(aww_ref[...], x, preferred_element_type=f32) + awb_ref[...]

    for row in range(_NH * 16):
        ch = 2 * row
        lo = lax.bitcast_convert_type(
            vpv[ch:ch + 1, :].astype(jnp.bfloat16), jnp.uint16).astype(jnp.uint32)
        hi = lax.bitcast_convert_type(
            vpv[ch + 1:ch + 2, :].astype(jnp.bfloat16), jnp.uint16).astype(jnp.uint32)
        word = lo | (hi << 16)
        tbl_ref[0, row:row + 1, :] = lax.bitcast_convert_type(word, jnp.int32)

    q = lax.broadcasted_iota(jnp.int32, (1, nq), 1)
    ci = (q >> 6).astype(f32) * (64.0 / 63.0)
    cj = (q & 63).astype(f32) * (64.0 / 63.0)

    for hh in range(_NH):
        a = [awr[hh * 4 + p:hh * 4 + p + 1, :] for p in range(_NP)]
        m = jnp.maximum(jnp.maximum(a[0], a[1]), jnp.maximum(a[2], a[3]))
        e = [jnp.exp(ap - m) for ap in a]
        den = e[0] + e[1] + e[2] + e[3]
        for p in range(_NP):
            ap = e[p] / den
            ox = offs[hh * 8 + 2 * p:hh * 8 + 2 * p + 1, :]
            oy = offs[hh * 8 + 2 * p + 1:hh * 8 + 2 * p + 2, :]
            px = ci + ox - 0.5
            py = cj + oy - 0.5
            x0 = jnp.floor(px)
            fx = px - x0
            y0 = jnp.floor(py)
            fy = py - y0
            for n, (dx, dy) in enumerate(((0, 0), (1, 0), (0, 1), (1, 1))):
                xf = x0 + float(dx)
                yf = y0 + float(dy)
                valid = ((xf >= 0.0) & (xf <= 63.0) & (yf >= 0.0) & (yf <= 63.0))
                xi = jnp.clip(xf, 0.0, 63.0).astype(jnp.int32)
                yi = jnp.clip(yf, 0.0, 63.0).astype(jnp.int32)
                wx = fx if dx else 1.0 - fx
                wy = fy if dy else 1.0 - fy
                r = hh * 16 + p * 4 + n
                idx_ref[0, r:r + 1, :] = yi * 64 + xi
                wgt_ref[0, r:r + 1, :] = wx * wy * ap * valid.astype(f32)


def _proj(x, vp_w, vp_b, so_w, so_b, aw_w, aw_b):
    b, c, nq = x.shape
    no = so_w.shape[0]
    na = aw_w.shape[0]
    return pl.pallas_call(
        _proj_body,
        grid=(b,),
        in_specs=[
            pl.BlockSpec((1, c, nq), lambda i: (i, 0, 0)),
            pl.BlockSpec((c, c), lambda i: (0, 0)),
            pl.BlockSpec((c, 1), lambda i: (0, 0)),
            pl.BlockSpec((no, c), lambda i: (0, 0)),
            pl.BlockSpec((no, 1), lambda i: (0, 0)),
            pl.BlockSpec((na, c), lambda i: (0, 0)),
            pl.BlockSpec((na, 1), lambda i: (0, 0)),
        ],
        out_specs=[
            pl.BlockSpec((1, _NH * 16, nq), lambda i: (i, 0, 0)),
            pl.BlockSpec((1, _NH * 16, nq), lambda i: (i, 0, 0)),
            pl.BlockSpec((1, _NH * 16, nq), lambda i: (i, 0, 0)),
        ],
        out_shape=[
            jax.ShapeDtypeStruct((b, _NH * 16, nq), jnp.int32),
            jax.ShapeDtypeStruct((b, _NH * 16, nq), jnp.int32),
            jax.ShapeDtypeStruct((b, _NH * 16, nq), jnp.float32),
        ],
    )(x, vp_w, vp_b, so_w, so_b, aw_w, aw_b)


def _sc_body(tbl_hbm, idx_hbm, wgt_hbm, out_hbm, tbl_v, idx_v, wgt_v, out_v):
    wid = lax.axis_index("s") * _NC + lax.axis_index("c")
    nq = 4096
    nchunk = nq // _CH
    nblk = _CH // 16

    pltpu.sync_copy(tbl_hbm.at[wid], tbl_v)

    def chunk_body(ck, _):
        pltpu.sync_copy(idx_hbm.at[wid, :, pl.ds(ck * _CH, _CH)], idx_v)
        pltpu.sync_copy(wgt_hbm.at[wid, :, pl.ds(ck * _CH, _CH)], wgt_v)

        @plsc.parallel_loop(0, nblk, unroll=4)
        def blk_body(qb):
            acc = [jnp.zeros((16,), jnp.float32) for _ in range(32)]
            for k in range(16):
                r = idx_v[k, pl.ds(qb * 16, 16)]
                w = wgt_v[k, pl.ds(qb * 16, 16)]
                for d2 in range(16):
                    g = plsc.load_gather(tbl_v, [r + d2 * 4096])
                    pair = plsc.bitcast(g, jnp.bfloat16)
                    glo, ghi = plsc.unpack(pair, format=plsc.PackFormat.INTERLEAVED)
                    acc[2 * d2] = acc[2 * d2] + w * glo
                    acc[2 * d2 + 1] = acc[2 * d2 + 1] + w * ghi
            for dd in range(32):
                out_v[dd, pl.ds(qb * 16, 16)] = acc[dd]

        pltpu.sync_copy(out_v, out_hbm.at[wid, :, pl.ds(ck * _CH, _CH)])
        return 0

    lax.fori_loop(0, nchunk, chunk_body, 0)


def _sc_sample(tbl_flat, idx_r, wgt_r):
    mesh = plsc.VectorSubcoreMesh(core_axis_name="c", subcore_axis_name="s")
    f = functools.partial(
        pl.kernel,
        out_type=jax.ShapeDtypeStruct((_NW, 32, 4096), jnp.float32),
        mesh=mesh,
        compiler_params=pltpu.CompilerParams(needs_layout_passes=False),
        scratch_types=[
            pltpu.VMEM((16 * 4096,), jnp.int32),
            pltpu.VMEM((16, _CH), jnp.int32),
            pltpu.VMEM((16, _CH), jnp.float32),
            pltpu.VMEM((32, _CH), jnp.float32),
        ],
    )(_sc_body)
    return f(tbl_flat, idx_r, wgt_r)


def _out_body(s_ref, x_ref, opw_ref, opb_ref, ow_ref, ob_ref, z_ref):
    f32 = jnp.float32
    o1 = (jnp.dot(opw_ref[...], s_ref[0], preferred_element_type=f32)
          + opb_ref[...] + x_ref[0])
    z_ref[0] = jnp.dot(ow_ref[...], o1, preferred_element_type=f32) + ob_ref[...]


def _out(s, x, op_w, op_b, out_w, out_b):
    b, c, nq = x.shape
    return pl.pallas_call(
        _out_body,
        grid=(b,),
        in_specs=[
            pl.BlockSpec((1, c, nq), lambda i: (i, 0, 0)),
            pl.BlockSpec((1, c, nq), lambda i: (i, 0, 0)),
            pl.BlockSpec((c, c), lambda i: (0, 0)),
            pl.BlockSpec((c, 1), lambda i: (0, 0)),
            pl.BlockSpec((c, c), lambda i: (0, 0)),
            pl.BlockSpec((c, 1), lambda i: (0, 0)),
        ],
        out_specs=pl.BlockSpec((1, c, nq), lambda i: (i, 0, 0)),
        out_shape=jax.ShapeDtypeStruct((b, c, nq), jnp.float32),
    )(s, x, op_w, op_b, out_w, out_b)


@jax.jit
def kernel(value, so_w, so_b, aw_w, aw_b, vp_w, vp_b, op_w, op_b, out_w, out_b):
    b, c, w, h = value.shape
    nq = w * h
    x = value.reshape(b, c, nq)
    tbl, idx, wgt = _proj(x, vp_w, vp_b.reshape(-1, 1), so_w, so_b.reshape(-1, 1),
                          aw_w, aw_b.reshape(-1, 1))
    tbl_flat = tbl.reshape(b * _NH, 16 * nq)
    idx_r = idx.reshape(b * _NH, 16, nq)
    wgt_r = wgt.reshape(b * _NH, 16, nq)
    smp = _sc_sample(tbl_flat, idx_r, wgt_r).reshape(b, c, nq)
    z = _out(smp, x, op_w, op_b.reshape(-1, 1), out_w, out_b.reshape(-1, 1))
    return z.reshape(b, c, w, h)


# speedup vs baseline: 1.0393x; 1.0393x over previous
import functools

import jax
import jax.numpy as jnp
from jax import lax
from jax.experimental import pallas as pl
from jax.experimental.pallas import tpu as pltpu
from jax.experimental.pallas import tpu_sc as plsc

_NH = 8
_NP = 4
_HD = 32
_NC = 2
_NS = 16
_NW = _NC * _NS
_CH = 512


def _proj_body(x_ref, vpw_ref, vpb_ref, sow_ref, sob_ref, aww_ref, awb_ref,
               tbl_ref, idx_ref, wgt_ref):
    x = x_ref[0]
    nq = x.shape[1]
    f32 = jnp.float32
    vpv = jnp.dot(vpw_ref[...], x, preferred_element_type=f32) + vpb_ref[...]
    offs = jnp.dot(sow_ref[...], x, preferred_element_type=f32) + sob_ref[...]
    awr = jnp.dot(aww_ref[...], x, preferred_element_type=f32) + awb_ref[...]

    for row in range(_NH * 16):
        ch = 2 * row
        lo = lax.bitcast_convert_type(
            vpv[ch:ch + 1, :].astype(jnp.bfloat16), jnp.uint16).astype(jnp.uint32)
        hi = lax.bitcast_convert_type(
            vpv[ch + 1:ch + 2, :].astype(jnp.bfloat16), jnp.uint16).astype(jnp.uint32)
        word = lo | (hi << 16)
        tbl_ref[0, row:row + 1, :] = lax.bitcast_convert_type(word, jnp.int32)

    q = lax.broadcasted_iota(jnp.int32, (1, nq), 1)
    ci = (q >> 6).astype(f32) * (64.0 / 63.0)
    cj = (q & 63).astype(f32) * (64.0 / 63.0)

    for hh in range(_NH):
        a = [awr[hh * 4 + p:hh * 4 + p + 1, :] for p in range(_NP)]
        m = jnp.maximum(jnp.maximum(a[0], a[1]), jnp.maximum(a[2], a[3]))
        e = [jnp.exp(ap - m) for ap in a]
        den = e[0] + e[1] + e[2] + e[3]
        for p in range(_NP):
            ap = e[p] / den
            ox = offs[hh * 8 + 2 * p:hh * 8 + 2 * p + 1, :]
            oy = offs[hh * 8 + 2 * p + 1:hh * 8 + 2 * p + 2, :]
            px = ci + ox - 0.5
            py = cj + oy - 0.5
            x0 = jnp.floor(px)
            fx = px - x0
            y0 = jnp.floor(py)
            fy = py - y0
            for n, (dx, dy) in enumerate(((0, 0), (1, 0), (0, 1), (1, 1))):
                xf = x0 + float(dx)
                yf = y0 + float(dy)
                valid = ((xf >= 0.0) & (xf <= 63.0) & (yf >= 0.0) & (yf <= 63.0))
                xi = jnp.clip(xf, 0.0, 63.0).astype(jnp.int32)
                yi = jnp.clip(yf, 0.0, 63.0).astype(jnp.int32)
                wx = fx if dx else 1.0 - fx
                wy = fy if dy else 1.0 - fy
                r = hh * 16 + p * 4 + n
                idx_ref[0, r:r + 1, :] = yi * 64 + xi
                wgt_ref[0, r:r + 1, :] = wx * wy * ap * valid.astype(f32)


def _proj(x, vp_w, vp_b, so_w, so_b, aw_w, aw_b):
    b, c, nq = x.shape
    no = so_w.shape[0]
    na = aw_w.shape[0]
    return pl.pallas_call(
        _proj_body,
        grid=(b,),
        in_specs=[
            pl.BlockSpec((1, c, nq), lambda i: (i, 0, 0)),
            pl.BlockSpec((c, c), lambda i: (0, 0)),
            pl.BlockSpec((c, 1), lambda i: (0, 0)),
            pl.BlockSpec((no, c), lambda i: (0, 0)),
            pl.BlockSpec((no, 1), lambda i: (0, 0)),
            pl.BlockSpec((na, c), lambda i: (0, 0)),
            pl.BlockSpec((na, 1), lambda i: (0, 0)),
        ],
        out_specs=[
            pl.BlockSpec((1, _NH * 16, nq), lambda i: (i, 0, 0)),
            pl.BlockSpec((1, _NH * 16, nq), lambda i: (i, 0, 0)),
            pl.BlockSpec((1, _NH * 16, nq), lambda i: (i, 0, 0)),
        ],
        out_shape=[
            jax.ShapeDtypeStruct((b, _NH * 16, nq), jnp.int32),
            jax.ShapeDtypeStruct((b, _NH * 16, nq), jnp.int32),
            jax.ShapeDtypeStruct((b, _NH * 16, nq), jnp.float32),
        ],
    )(x, vp_w, vp_b, so_w, so_b, aw_w, aw_b)


def _sc_body(tbl_hbm, idx_hbm, wgt_hbm, out_hbm, tbl_v, idx_v, wgt_v, out_v):
    wid = lax.axis_index("s") * _NC + lax.axis_index("c")
    nq = 4096
    nchunk = nq // _CH
    nblk = _CH // 16

    pltpu.sync_copy(tbl_hbm.at[wid], tbl_v)

    def chunk_body(ck, _):
        pltpu.sync_copy(idx_hbm.at[wid, :, pl.ds(ck * _CH, _CH)], idx_v)
        pltpu.sync_copy(wgt_hbm.at[wid, :, pl.ds(ck * _CH, _CH)], wgt_v)

        @plsc.parallel_loop(0, nblk, unroll=1)
        def blk_body(qb):
            acc = [jnp.zeros((16,), jnp.float32) for _ in range(32)]
            for k in range(16):
                r = idx_v[k, pl.ds(qb * 16, 16)]
                w = wgt_v[k, pl.ds(qb * 16, 16)]
                for d2 in range(16):
                    g = plsc.load_gather(tbl_v, [r + d2 * 4096])
                    pair = plsc.bitcast(g, jnp.bfloat16)
                    glo, ghi = plsc.unpack(pair, format=plsc.PackFormat.INTERLEAVED)
                    acc[2 * d2] = acc[2 * d2] + w * glo
                    acc[2 * d2 + 1] = acc[2 * d2 + 1] + w * ghi
            for dd in range(32):
                out_v[dd, pl.ds(qb * 16, 16)] = acc[dd]

        pltpu.sync_copy(out_v, out_hbm.at[wid, :, pl.ds(ck * _CH, _CH)])
        return 0

    lax.fori_loop(0, nchunk, chunk_body, 0)


def _sc_sample(tbl_flat, idx_r, wgt_r):
    mesh = plsc.VectorSubcoreMesh(core_axis_name="c", subcore_axis_name="s")
    f = functools.partial(
        pl.kernel,
        out_type=jax.ShapeDtypeStruct((_NW, 32, 4096), jnp.float32),
        mesh=mesh,
        compiler_params=pltpu.CompilerParams(needs_layout_passes=False),
        scratch_types=[
            pltpu.VMEM((16 * 4096,), jnp.int32),
            pltpu.VMEM((16, _CH), jnp.int32),
            pltpu.VMEM((16, _CH), jnp.float32),
            pltpu.VMEM((32, _CH), jnp.float32),
        ],
    )(_sc_body)
    return f(tbl_flat, idx_r, wgt_r)


def _out_body(s_ref, x_ref, opw_ref, opb_ref, ow_ref, ob_ref, z_ref):
    f32 = jnp.float32
    o1 = (jnp.dot(opw_ref[...], s_ref[0], preferred_element_type=f32)
          + opb_ref[...] + x_ref[0])
    z_ref[0] = jnp.dot(ow_ref[...], o1, preferred_element_type=f32) + ob_ref[...]


def _out(s, x, op_w, op_b, out_w, out_b):
    b, c, nq = x.shape
    return pl.pallas_call(
        _out_body,
        grid=(b,),
        in_specs=[
            pl.BlockSpec((1, c, nq), lambda i: (i, 0, 0)),
            pl.BlockSpec((1, c, nq), lambda i: (i, 0, 0)),
            pl.BlockSpec((c, c), lambda i: (0, 0)),
            pl.BlockSpec((c, 1), lambda i: (0, 0)),
            pl.BlockSpec((c, c), lambda i: (0, 0)),
            pl.BlockSpec((c, 1), lambda i: (0, 0)),
        ],
        out_specs=pl.BlockSpec((1, c, nq), lambda i: (i, 0, 0)),
        out_shape=jax.ShapeDtypeStruct((b, c, nq), jnp.float32),
    )(s, x, op_w, op_b, out_w, out_b)


@jax.jit
def kernel(value, so_w, so_b, aw_w, aw_b, vp_w, vp_b, op_w, op_b, out_w, out_b):
    b, c, w, h = value.shape
    nq = w * h
    x = value.reshape(b, c, nq)
    tbl, idx, wgt = _proj(x, vp_w, vp_b.reshape(-1, 1), so_w, so_b.reshape(-1, 1),
                          aw_w, aw_b.reshape(-1, 1))
    tbl_flat = tbl.reshape(b * _NH, 16 * nq)
    idx_r = idx.reshape(b * _NH, 16, nq)
    wgt_r = wgt.reshape(b * _NH, 16, nq)
    smp = _sc_sample(tbl_flat, idx_r, wgt_r).reshape(b, c, nq)
    z = _out(smp, x, op_w, op_b.reshape(-1, 1), out_w, out_b.reshape(-1, 1))
    return z.reshape(b, c, w, h)


# speedup vs baseline: 1.0552x; 1.0153x over previous
import functools

import jax
import jax.numpy as jnp
from jax import lax
from jax.experimental import pallas as pl
from jax.experimental.pallas import tpu as pltpu
from jax.experimental.pallas import tpu_sc as plsc

_NH = 8
_NP = 4
_HD = 32
_NC = 2
_NS = 16
_NW = _NC * _NS
_CH = 512


def _proj_body(x_ref, vpw_ref, vpb_ref, sow_ref, sob_ref, aww_ref, awb_ref,
               tbl_ref, idx_ref, wgt_ref):
    x = x_ref[0]
    nq = x.shape[1]
    f32 = jnp.float32
    vpv = jnp.dot(vpw_ref[...], x, preferred_element_type=f32) + vpb_ref[...]
    offs = jnp.dot(sow_ref[...], x, preferred_element_type=f32) + sob_ref[...]
    awr = jnp.dot(aww_ref[...], x, preferred_element_type=f32) + awb_ref[...]

    for row in range(_NH * 16):
        ch = 2 * row
        lo = lax.bitcast_convert_type(
            vpv[ch:ch + 1, :].astype(jnp.bfloat16), jnp.uint16).astype(jnp.uint32)
        hi = lax.bitcast_convert_type(
            vpv[ch + 1:ch + 2, :].astype(jnp.bfloat16), jnp.uint16).astype(jnp.uint32)
        word = lo | (hi << 16)
        tbl_ref[0, row:row + 1, :] = lax.bitcast_convert_type(word, jnp.int32)

    q = lax.broadcasted_iota(jnp.int32, (1, nq), 1)
    ci = (q >> 6).astype(f32) * (64.0 / 63.0)
    cj = (q & 63).astype(f32) * (64.0 / 63.0)

    for hh in range(_NH):
        a = [awr[hh * 4 + p:hh * 4 + p + 1, :] for p in range(_NP)]
        m = jnp.maximum(jnp.maximum(a[0], a[1]), jnp.maximum(a[2], a[3]))
        e = [jnp.exp(ap - m) for ap in a]
        den = e[0] + e[1] + e[2] + e[3]
        for p in range(_NP):
            ap = e[p] / den
            ox = offs[hh * 8 + 2 * p:hh * 8 + 2 * p + 1, :]
            oy = offs[hh * 8 + 2 * p + 1:hh * 8 + 2 * p + 2, :]
            px = ci + ox - 0.5
            py = cj + oy - 0.5
            x0 = jnp.floor(px)
            fx = px - x0
            y0 = jnp.floor(py)
            fy = py - y0
            for n, (dx, dy) in enumerate(((0, 0), (1, 0), (0, 1), (1, 1))):
                xf = x0 + float(dx)
                yf = y0 + float(dy)
                valid = ((xf >= 0.0) & (xf <= 63.0) & (yf >= 0.0) & (yf <= 63.0))
                xi = jnp.clip(xf, 0.0, 63.0).astype(jnp.int32)
                yi = jnp.clip(yf, 0.0, 63.0).astype(jnp.int32)
                wx = fx if dx else 1.0 - fx
                wy = fy if dy else 1.0 - fy
                r = hh * 16 + p * 4 + n
                idx_ref[0, r:r + 1, :] = yi * 64 + xi
                wgt_ref[0, r:r + 1, :] = wx * wy * ap * valid.astype(f32)


def _proj(x, vp_w, vp_b, so_w, so_b, aw_w, aw_b):
    b, c, nq = x.shape
    no = so_w.shape[0]
    na = aw_w.shape[0]
    return pl.pallas_call(
        _proj_body,
        grid=(b,),
        in_specs=[
            pl.BlockSpec((1, c, nq), lambda i: (i, 0, 0)),
            pl.BlockSpec((c, c), lambda i: (0, 0)),
            pl.BlockSpec((c, 1), lambda i: (0, 0)),
            pl.BlockSpec((no, c), lambda i: (0, 0)),
            pl.BlockSpec((no, 1), lambda i: (0, 0)),
            pl.BlockSpec((na, c), lambda i: (0, 0)),
            pl.BlockSpec((na, 1), lambda i: (0, 0)),
        ],
        out_specs=[
            pl.BlockSpec((1, _NH * 16, nq), lambda i: (i, 0, 0)),
            pl.BlockSpec((1, _NH * 16, nq), lambda i: (i, 0, 0)),
            pl.BlockSpec((1, _NH * 16, nq), lambda i: (i, 0, 0)),
        ],
        out_shape=[
            jax.ShapeDtypeStruct((b, _NH * 16, nq), jnp.int32),
            jax.ShapeDtypeStruct((b, _NH * 16, nq), jnp.int32),
            jax.ShapeDtypeStruct((b, _NH * 16, nq), jnp.float32),
        ],
    )(x, vp_w, vp_b, so_w, so_b, aw_w, aw_b)


def _sc_body(tbl_hbm, idx_hbm, wgt_hbm, out_hbm, tbl_v, idx_v, wgt_v, out_v):
    wid = lax.axis_index("s") * _NC + lax.axis_index("c")
    nq = 4096
    nchunk = nq // _CH
    nblk = _CH // 16

    pltpu.sync_copy(tbl_hbm.at[wid], tbl_v)

    def chunk_body(ck, _):
        pltpu.sync_copy(idx_hbm.at[wid, :, pl.ds(ck * _CH, _CH)], idx_v)
        pltpu.sync_copy(wgt_hbm.at[wid, :, pl.ds(ck * _CH, _CH)], wgt_v)

        @plsc.parallel_loop(0, nblk, unroll=2)
        def blk_body(qb):
            acc = [jnp.zeros((16,), jnp.float32) for _ in range(32)]
            for k in range(16):
                r = idx_v[k, pl.ds(qb * 16, 16)]
                w = wgt_v[k, pl.ds(qb * 16, 16)]
                for d2 in range(16):
                    g = plsc.load_gather(tbl_v, [r + d2 * 4096])
                    pair = plsc.bitcast(g, jnp.bfloat16)
                    glo, ghi = plsc.unpack(pair, format=plsc.PackFormat.INTERLEAVED)
                    acc[2 * d2] = acc[2 * d2] + w * glo
                    acc[2 * d2 + 1] = acc[2 * d2 + 1] + w * ghi
            for dd in range(32):
                out_v[dd, pl.ds(qb * 16, 16)] = acc[dd]

        pltpu.sync_copy(out_v, out_hbm.at[wid, :, pl.ds(ck * _CH, _CH)])
        return 0

    lax.fori_loop(0, nchunk, chunk_body, 0)


def _sc_sample(tbl_flat, idx_r, wgt_r):
    mesh = plsc.VectorSubcoreMesh(core_axis_name="c", subcore_axis_name="s")
    f = functools.partial(
        pl.kernel,
        out_type=jax.ShapeDtypeStruct((_NW, 32, 4096), jnp.float32),
        mesh=mesh,
        compiler_params=pltpu.CompilerParams(needs_layout_passes=False),
        scratch_types=[
            pltpu.VMEM((16 * 4096,), jnp.int32),
            pltpu.VMEM((16, _CH), jnp.int32),
            pltpu.VMEM((16, _CH), jnp.float32),
            pltpu.VMEM((32, _CH), jnp.float32),
        ],
    )(_sc_body)
    return f(tbl_flat, idx_r, wgt_r)


def _out_body(s_ref, x_ref, opw_ref, opb_ref, ow_ref, ob_ref, z_ref):
    f32 = jnp.float32
    o1 = (jnp.dot(opw_ref[...], s_ref[0], preferred_element_type=f32)
          + opb_ref[...] + x_ref[0])
    z_ref[0] = jnp.dot(ow_ref[...], o1, preferred_element_type=f32) + ob_ref[...]


def _out(s, x, op_w, op_b, out_w, out_b):
    b, c, nq = x.shape
    return pl.pallas_call(
        _out_body,
        grid=(b,),
        in_specs=[
            pl.BlockSpec((1, c, nq), lambda i: (i, 0, 0)),
            pl.BlockSpec((1, c, nq), lambda i: (i, 0, 0)),
            pl.BlockSpec((c, c), lambda i: (0, 0)),
            pl.BlockSpec((c, 1), lambda i: (0, 0)),
            pl.BlockSpec((c, c), lambda i: (0, 0)),
            pl.BlockSpec((c, 1), lambda i: (0, 0)),
        ],
        out_specs=pl.BlockSpec((1, c, nq), lambda i: (i, 0, 0)),
        out_shape=jax.ShapeDtypeStruct((b, c, nq), jnp.float32),
    )(s, x, op_w, op_b, out_w, out_b)


@jax.jit
def kernel(value, so_w, so_b, aw_w, aw_b, vp_w, vp_b, op_w, op_b, out_w, out_b):
    b, c, w, h = value.shape
    nq = w * h
    x = value.reshape(b, c, nq)
    tbl, idx, wgt = _proj(x, vp_w, vp_b.reshape(-1, 1), so_w, so_b.reshape(-1, 1),
                          aw_w, aw_b.reshape(-1, 1))
    tbl_flat = tbl.reshape(b * _NH, 16 * nq)
    idx_r = idx.reshape(b * _NH, 16, nq)
    wgt_r = wgt.reshape(b * _NH, 16, nq)
    smp = _sc_sample(tbl_flat, idx_r, wgt_r).reshape(b, c, nq)
    z = _out(smp, x, op_w, op_b.reshape(-1, 1), out_w, out_b.reshape(-1, 1))
    return z.reshape(b, c, w, h)
